# Initial kernel scaffold; baseline (speedup 1.0000x reference)
import jax
import jax.numpy as jnp
from jax.experimental import pallas as pl


def kernel(x, edge_index, edge_type, W1, b1, w1, root1, bias1, w2, root2, bias2):
    raise NotImplementedError("write your pallas kernel here")


# speedup vs baseline: 1.6634x; 1.6634x over previous
import functools

import jax
import jax.numpy as jnp
from jax import lax
from jax.experimental import pallas as pl
from jax.experimental.pallas import tpu as pltpu
from jax.experimental.pallas import tpu_sc as plsc

N = 10000
E = 160000
R = 8
RR = R + 1
D = 256
DH = 128
NSUB = 16
EPS = E // NSUB
C = 80
NCHUNK = EPS // C
SEGS = N * R
SEGROWS = 40960
TRASH = 40944


def _mesh():
    return plsc.VectorSubcoreMesh(core_axis_name="c", subcore_axis_name="s")


def _mm_body(x_ref, w_ref, b_ref, o_ref):
    o_ref[...] = jnp.dot(x_ref[...], w_ref[...],
                         preferred_element_type=jnp.float32) + b_ref[...]


def _linear(x, W, b, bm):
    n, k = x.shape
    m = W.shape[1]
    return pl.pallas_call(
        _mm_body,
        grid=(n // bm,),
        in_specs=[
            pl.BlockSpec((bm, k), lambda i: (i, 0)),
            pl.BlockSpec((k, m), lambda i: (0, 0)),
            pl.BlockSpec((1, m), lambda i: (0, 0)),
        ],
        out_specs=pl.BlockSpec((bm, m), lambda i: (i, 0)),
        out_shape=jax.ShapeDtypeStruct((n, m), jnp.float32),
    )(x, W, b.reshape(1, m))


def _hcat_body(relu_in, h_ref, w_ref, b_ref, o_ref):
    h = h_ref[...]
    if relu_in:
        h = jnp.maximum(h, 0.0)
    b = jnp.where(pl.program_id(0) == 0, b_ref[0, 0:1, :], b_ref[0, 1:2, :])
    o_ref[...] = jnp.dot(h, w_ref[0], preferred_element_type=jnp.float32) + b


def _hcat(h, Wext, bext, relu_in):
    n, k = h.shape
    bm = 1000
    return pl.pallas_call(
        functools.partial(_hcat_body, relu_in),
        grid=(2, RR, n // bm),
        in_specs=[
            pl.BlockSpec((bm, k), lambda c, r, i: (i, 0)),
            pl.BlockSpec((1, k, DH), lambda c, r, i: (r, 0, c)),
            pl.BlockSpec((1, 2, DH), lambda c, r, i: (r, 0, 0)),
        ],
        out_specs=pl.BlockSpec((bm, DH),
                               lambda c, r, i: (c * RR * (n // bm) + r * (n // bm) + i, 0)),
        out_shape=jax.ShapeDtypeStruct((2 * RR * N, DH), jnp.float32),
    )(h, Wext, bext)


CE = 2000
NSEG = 81920
PRANGE = 8192
PTRASH = PRANGE

@functools.partial(
    pl.kernel,
    mesh=_mesh(),
    out_type=jax.ShapeDtypeStruct((NSEG, 128), jnp.float32),
    scratch_types=[
        pltpu.VMEM_SHARED((PRANGE + 16, 128), jnp.float32),
        pltpu.VMEM((80, 128), jnp.float32),
        pltpu.VMEM((64, 128), jnp.float32),
        pltpu.VMEM((64, 128), jnp.float32),
        pltpu.VMEM((CE,), jnp.int32),
        pltpu.VMEM((CE,), jnp.int32),
        pltpu.VMEM((25, 80), jnp.int32),
        pltpu.SemaphoreType.DMA,
    ],
)
def _count_kernel(dst_hbm, typ_hbm, invw_hbm, cnt_sp, ones_b, rbuf, obuf,
                  dstb, typb, segb, sem):
    cid = lax.axis_index("c")
    sid = lax.axis_index("s")
    zero16 = jnp.zeros((16,), jnp.float32)
    one16 = jnp.ones((16,), jnp.float32)

    def fill(i, _):
        for k in range(8):
            ones_b[i, pl.ds(k * 16, 16)] = one16
        return 0

    lax.fori_loop(0, 80, fill, 0)

    def one_pass(p, _):
        b0 = (2 * p + cid) * PRANGE
        def zf(i, _):
            for k in range(8):
                rbuf[i, pl.ds(k * 16, 16)] = zero16
            return 0

        lax.fori_loop(0, 64, zf, 0)
        for q in range(8):
            pltpu.sync_copy(rbuf, cnt_sp.at[pl.ds(sid * 512 + q * 64, 64)])

        @pl.when(sid == 0)
        def _():
            pltpu.sync_copy(rbuf.at[pl.ds(0, 16)], cnt_sp.at[pl.ds(PRANGE, 16)])
        plsc.subcore_barrier()

        def chunk(c, _):
            base = sid * EPS + c * CE
            pltpu.sync_copy(dst_hbm.at[pl.ds(base, CE)], dstb)
            pltpu.sync_copy(typ_hbm.at[pl.ds(base, CE)], typb)
            for m in range(25):
                for j in range(5):
                    off = pl.ds(m * 80 + j * 16, 16)
                    seg = dstb[off] * R + typb[off] - b0
                    inr = (seg >= 0) & (seg < PRANGE)
                    segb[m, pl.ds(j * 16, 16)] = jnp.where(inr, seg, PTRASH)
            handles = [pltpu.async_copy(ones_b, cnt_sp.at[segb.at[m]], sem,
                                        add=True) for m in range(25)]
            for h in handles:
                h.wait()
            return 0

        lax.fori_loop(0, EPS // CE, chunk, 0)
        plsc.subcore_barrier()

        def inv_q(q, _):
            r0 = sid * 512 + q * 64
            pltpu.sync_copy(cnt_sp.at[pl.ds(r0, 64)], rbuf)

            def inv(i, _):
                v = 1.0 / jnp.maximum(rbuf[i, pl.ds(0, 16)], 1.0)
                for k in range(8):
                    obuf[i, pl.ds(k * 16, 16)] = v
                return 0

            lax.fori_loop(0, 64, inv, 0)
            pltpu.sync_copy(obuf, invw_hbm.at[pl.ds(b0 + r0, 64)])
            return 0

        lax.fori_loop(0, 8, inv_q, 0)
        plsc.subcore_barrier()
        return 0

    lax.fori_loop(0, 5, one_pass, 0)


@functools.partial(
    pl.kernel,
    mesh=_mesh(),
    out_type=jax.ShapeDtypeStruct((2 * N, DH), jnp.float32),
    scratch_types=[
        pltpu.VMEM_SHARED((N, DH), jnp.float32),
        pltpu.VMEM((40, DH), jnp.float32),
        pltpu.VMEM((40, DH), jnp.float32),
        pltpu.VMEM((C, DH), jnp.float32),
        pltpu.VMEM((C, 128), jnp.float32),
        pltpu.VMEM((CE,), jnp.int32),
        pltpu.VMEM((CE,), jnp.int32),
        pltpu.VMEM((CE,), jnp.int32),
        pltpu.VMEM((C,), jnp.int32),
        pltpu.VMEM((C,), jnp.int32),
        pltpu.VMEM((C,), jnp.int32),
    ],
)
def _agg_kernel(hcat, src_hbm, dst_hbm, typ_hbm, invw, y_hbm,
                acc, zbuf, pbuf, rows, wbuf, srcb, dstb, typb, gb, segb, db80):
    cid = lax.axis_index("c")
    sid = lax.axis_index("s")
    zero16f = jnp.zeros((16,), jnp.float32)

    def zfill(i, _):
        for k in range(DH // 16):
            zbuf[i, pl.ds(k * 16, 16)] = zero16f
        return 0

    lax.fori_loop(0, 40, zfill, 0)
    def zc(m, _):
        idx = sid + NSUB * m

        @pl.when(idx < 250)
        def _():
            pltpu.sync_copy(zbuf, acc.at[pl.ds(idx * 40, 40)])
        return 0

    lax.fori_loop(0, 16, zc, 0)
    plsc.subcore_barrier()

    def macro(i, _):
        base = sid * EPS + i * CE
        pltpu.sync_copy(src_hbm.at[pl.ds(base, CE)], srcb)
        pltpu.sync_copy(dst_hbm.at[pl.ds(base, CE)], dstb)
        pltpu.sync_copy(typ_hbm.at[pl.ds(base, CE)], typb)

        def sub(m, _):
            for j in range(C // 16):
                off = pl.ds(m * C + j * 16, 16)
                s16 = srcb[off]
                t16 = typb[off]
                d16 = dstb[off]
                gb[pl.ds(j * 16, 16)] = cid * (RR * N) + t16 * N + s16
                segb[pl.ds(j * 16, 16)] = d16 * R + t16
                db80[pl.ds(j * 16, 16)] = d16
            pltpu.sync_copy(invw.at[segb], wbuf)
            pltpu.sync_copy(hcat.at[gb], rows)

            def scale(e, _):
                ws = wbuf[e, pl.ds(0, 16)][0]
                for k in range(DH // 16):
                    rows[e, pl.ds(k * 16, 16)] = rows[e, pl.ds(k * 16, 16)] * ws
                return 0

            lax.fori_loop(0, C, scale, 0)
            pltpu.sync_copy(rows, acc.at[db80], add=True)
            return 0

        lax.fori_loop(0, CE // C, sub, 0)
        return 0

    lax.fori_loop(0, EPS // CE, macro, 0)
    plsc.subcore_barrier()

    def fl(m, _):
        idx = sid + NSUB * m

        @pl.when(idx < 250)
        def _():
            r0 = idx * 40
            pltpu.sync_copy(acc.at[pl.ds(r0, 40)], zbuf)
            pltpu.sync_copy(hcat.at[pl.ds(cid * (RR * N) + R * N + r0, 40)], pbuf)

            def addp(i, _):
                for k in range(DH // 16):
                    zbuf[i, pl.ds(k * 16, 16)] = (zbuf[i, pl.ds(k * 16, 16)] +
                                                  pbuf[i, pl.ds(k * 16, 16)])
                return 0

            lax.fori_loop(0, 40, addp, 0)
            pltpu.sync_copy(zbuf, y_hbm.at[pl.ds(cid * N + r0, 40)])
        return 0

    lax.fori_loop(0, 16, fl, 0)


def _build_weights(w, root, bias, k_pad):
    r, nb, bin_, bout = w.shape
    Wbd = jnp.zeros((r, k_pad, D), jnp.float32)
    for ri in range(r):
        for b in range(nb):
            Wbd = Wbd.at[ri, b * bin_:(b + 1) * bin_, b * bout:(b + 1) * bout].set(w[ri, b])
    rootp = jnp.zeros((k_pad, D), jnp.float32).at[:root.shape[0], :root.shape[1]].set(root)
    Wext = jnp.concatenate([Wbd, rootp[None]], axis=0)
    bext = jnp.zeros((RR, 2, DH), jnp.float32)
    bext = bext.at[R].set(jnp.pad(bias, (0, D - bias.shape[0])).reshape(2, DH))
    return Wext, bext


def kernel(x, edge_index, edge_type, W1, b1, w1, root1, bias1, w2, root2, bias2):
    src = edge_index[0]
    dst = edge_index[1]
    typ = edge_type

    h1 = _linear(x, W1, b1, bm=1000)

    Wext1, bext1 = _build_weights(w1, root1, bias1, k_pad=500)
    Hcat1 = _hcat(h1, Wext1, bext1, relu_in=False)

    invw = _count_kernel(dst, typ)

    Y1 = _agg_kernel(Hcat1, src, dst, typ, invw)
    h2 = jnp.concatenate([Y1[:N], Y1[N:]], axis=1)

    Wext2, bext2 = _build_weights(w2, root2, bias2, k_pad=D)
    Hcat2 = _hcat(h2, Wext2, bext2, relu_in=True)

    Y2 = _agg_kernel(Hcat2, src, dst, typ, invw)
    return jnp.concatenate([Y2[:N], Y2[N:]], axis=1)[:, :250]


# speedup vs baseline: 1.8330x; 1.1020x over previous
import functools

import jax
import jax.numpy as jnp
from jax import lax
from jax.experimental import pallas as pl
from jax.experimental.pallas import tpu as pltpu
from jax.experimental.pallas import tpu_sc as plsc

N = 10000
E = 160000
R = 8
RR = R + 1
D = 256
DH = 128
NSUB = 16
EPS = E // NSUB
C = 80
NCHUNK = EPS // C
SEGS = N * R
SEGROWS = 40960
TRASH = 40944


def _mesh():
    return plsc.VectorSubcoreMesh(core_axis_name="c", subcore_axis_name="s")


def _mm_body(x_ref, w_ref, b_ref, o_ref):
    o_ref[...] = jnp.dot(x_ref[...], w_ref[...],
                         preferred_element_type=jnp.float32) + b_ref[...]


def _linear(x, W, b, bm):
    n, k = x.shape
    m = W.shape[1]
    return pl.pallas_call(
        _mm_body,
        grid=(n // bm,),
        in_specs=[
            pl.BlockSpec((bm, k), lambda i: (i, 0)),
            pl.BlockSpec((k, m), lambda i: (0, 0)),
            pl.BlockSpec((1, m), lambda i: (0, 0)),
        ],
        out_specs=pl.BlockSpec((bm, m), lambda i: (i, 0)),
        out_shape=jax.ShapeDtypeStruct((n, m), jnp.float32),
    )(x, W, b.reshape(1, m))


def _hcat_body(relu_in, h_ref, w_ref, b_ref, o_ref):
    h = h_ref[...]
    if relu_in:
        h = jnp.maximum(h, 0.0)
    b = jnp.where(pl.program_id(0) == 0, b_ref[0, 0:1, :], b_ref[0, 1:2, :])
    o_ref[...] = jnp.dot(h, w_ref[0], preferred_element_type=jnp.float32) + b


def _hcat(h, Wext, bext, relu_in):
    n, k = h.shape
    bm = 1000
    return pl.pallas_call(
        functools.partial(_hcat_body, relu_in),
        grid=(2, RR, n // bm),
        in_specs=[
            pl.BlockSpec((bm, k), lambda c, r, i: (i, 0)),
            pl.BlockSpec((1, k, DH), lambda c, r, i: (r, 0, c)),
            pl.BlockSpec((1, 2, DH), lambda c, r, i: (r, 0, 0)),
        ],
        out_specs=pl.BlockSpec((bm, DH),
                               lambda c, r, i: (c * RR * (n // bm) + r * (n // bm) + i, 0)),
        out_shape=jax.ShapeDtypeStruct((2 * RR * N, DH), jnp.float32),
    )(h, Wext, bext)


CE = 2000
NSEG = 81920
PRANGE = 8192
PTRASH = PRANGE

@functools.partial(
    pl.kernel,
    mesh=_mesh(),
    out_type=jax.ShapeDtypeStruct((NSEG, 128), jnp.float32),
    scratch_types=[
        pltpu.VMEM_SHARED((PRANGE + 16, 128), jnp.float32),
        pltpu.VMEM((80, 128), jnp.float32),
        pltpu.VMEM((64, 128), jnp.float32),
        pltpu.VMEM((64, 128), jnp.float32),
        pltpu.VMEM((CE,), jnp.int32),
        pltpu.VMEM((CE,), jnp.int32),
        pltpu.VMEM((25, 80), jnp.int32),
        pltpu.SemaphoreType.DMA,
    ],
)
def _count_kernel(dst_hbm, typ_hbm, invw_hbm, cnt_sp, ones_b, rbuf, obuf,
                  dstb, typb, segb, sem):
    cid = lax.axis_index("c")
    sid = lax.axis_index("s")
    zero16 = jnp.zeros((16,), jnp.float32)
    one16 = jnp.ones((16,), jnp.float32)

    def fill(i, _):
        for k in range(8):
            ones_b[i, pl.ds(k * 16, 16)] = one16
        return 0

    lax.fori_loop(0, 80, fill, 0)

    def one_pass(p, _):
        b0 = (2 * p + cid) * PRANGE
        def zf(i, _):
            for k in range(8):
                rbuf[i, pl.ds(k * 16, 16)] = zero16
            return 0

        lax.fori_loop(0, 64, zf, 0)
        for q in range(8):
            pltpu.sync_copy(rbuf, cnt_sp.at[pl.ds(sid * 512 + q * 64, 64)])

        @pl.when(sid == 0)
        def _():
            pltpu.sync_copy(rbuf.at[pl.ds(0, 16)], cnt_sp.at[pl.ds(PRANGE, 16)])
        plsc.subcore_barrier()

        def chunk(c, _):
            base = sid * EPS + c * CE
            pltpu.sync_copy(dst_hbm.at[pl.ds(base, CE)], dstb)
            pltpu.sync_copy(typ_hbm.at[pl.ds(base, CE)], typb)
            for m in range(25):
                for j in range(5):
                    off = pl.ds(m * 80 + j * 16, 16)
                    seg = dstb[off] * R + typb[off] - b0
                    inr = (seg >= 0) & (seg < PRANGE)
                    segb[m, pl.ds(j * 16, 16)] = jnp.where(inr, seg, PTRASH)
            handles = [pltpu.async_copy(ones_b, cnt_sp.at[segb.at[m]], sem,
                                        add=True) for m in range(25)]
            for h in handles:
                h.wait()
            return 0

        lax.fori_loop(0, EPS // CE, chunk, 0)
        plsc.subcore_barrier()

        def inv_q(q, _):
            r0 = sid * 512 + q * 64
            pltpu.sync_copy(cnt_sp.at[pl.ds(r0, 64)], rbuf)

            def inv(i, _):
                v = 1.0 / jnp.maximum(rbuf[i, pl.ds(0, 16)], 1.0)
                for k in range(8):
                    obuf[i, pl.ds(k * 16, 16)] = v
                return 0

            lax.fori_loop(0, 64, inv, 0)
            pltpu.sync_copy(obuf, invw_hbm.at[pl.ds(b0 + r0, 64)])
            return 0

        lax.fori_loop(0, 8, inv_q, 0)
        plsc.subcore_barrier()
        return 0

    lax.fori_loop(0, 5, one_pass, 0)


@functools.partial(
    pl.kernel,
    mesh=_mesh(),
    out_type=jax.ShapeDtypeStruct((2 * N, DH), jnp.float32),
    scratch_types=[
        pltpu.VMEM_SHARED((N, DH), jnp.float32),
        pltpu.VMEM((40, DH), jnp.float32),
        pltpu.VMEM((40, DH), jnp.float32),
        pltpu.VMEM((C, DH), jnp.float32),
        pltpu.VMEM((C, 128), jnp.float32),
        pltpu.VMEM((CE,), jnp.int32),
        pltpu.VMEM((CE,), jnp.int32),
        pltpu.VMEM((CE,), jnp.int32),
        pltpu.VMEM((C,), jnp.int32),
        pltpu.VMEM((C,), jnp.int32),
        pltpu.VMEM((C,), jnp.int32),
        pltpu.SemaphoreType.DMA,
        pltpu.SemaphoreType.DMA,
    ],
)
def _agg_kernel(hcat, src_hbm, dst_hbm, typ_hbm, invw, y_hbm,
                acc, zbuf, pbuf, rows, wbuf, srcb, dstb, typb, gb, segb, db80,
                sem1, sem2):
    cid = lax.axis_index("c")
    sid = lax.axis_index("s")
    zero16f = jnp.zeros((16,), jnp.float32)

    def zfill(i, _):
        for k in range(DH // 16):
            zbuf[i, pl.ds(k * 16, 16)] = zero16f
        return 0

    lax.fori_loop(0, 40, zfill, 0)
    def zc(m, _):
        idx = sid + NSUB * m

        @pl.when(idx < 250)
        def _():
            pltpu.sync_copy(zbuf, acc.at[pl.ds(idx * 40, 40)])
        return 0

    lax.fori_loop(0, 16, zc, 0)
    plsc.subcore_barrier()

    def macro(i, _):
        base = sid * EPS + i * CE
        pltpu.sync_copy(src_hbm.at[pl.ds(base, CE)], srcb)
        pltpu.sync_copy(dst_hbm.at[pl.ds(base, CE)], dstb)
        pltpu.sync_copy(typ_hbm.at[pl.ds(base, CE)], typb)

        def sub(m, _):
            for j in range(C // 16):
                off = pl.ds(m * C + j * 16, 16)
                s16 = srcb[off]
                t16 = typb[off]
                d16 = dstb[off]
                gb[pl.ds(j * 16, 16)] = cid * (RR * N) + t16 * N + s16
                segb[pl.ds(j * 16, 16)] = d16 * R + t16
                db80[pl.ds(j * 16, 16)] = d16
            hw = pltpu.async_copy(invw.at[segb], wbuf, sem1)
            hr = pltpu.async_copy(hcat.at[gb], rows, sem2)
            hw.wait()
            hr.wait()

            def scale(e, _):
                ws = wbuf[e, pl.ds(0, 16)][0]
                for k in range(DH // 16):
                    rows[e, pl.ds(k * 16, 16)] = rows[e, pl.ds(k * 16, 16)] * ws
                return 0

            lax.fori_loop(0, C, scale, 0)
            pltpu.sync_copy(rows, acc.at[db80], add=True)
            return 0

        lax.fori_loop(0, CE // C, sub, 0)
        return 0

    lax.fori_loop(0, EPS // CE, macro, 0)
    plsc.subcore_barrier()

    def fl(m, _):
        idx = sid + NSUB * m

        @pl.when(idx < 250)
        def _():
            r0 = idx * 40
            pltpu.sync_copy(acc.at[pl.ds(r0, 40)], zbuf)
            pltpu.sync_copy(hcat.at[pl.ds(cid * (RR * N) + R * N + r0, 40)], pbuf)

            def addp(i, _):
                for k in range(DH // 16):
                    zbuf[i, pl.ds(k * 16, 16)] = (zbuf[i, pl.ds(k * 16, 16)] +
                                                  pbuf[i, pl.ds(k * 16, 16)])
                return 0

            lax.fori_loop(0, 40, addp, 0)
            pltpu.sync_copy(zbuf, y_hbm.at[pl.ds(cid * N + r0, 40)])
        return 0

    lax.fori_loop(0, 16, fl, 0)


def _build_weights(w, root, bias, k_pad):
    r, nb, bin_, bout = w.shape
    Wbd = jnp.zeros((r, k_pad, D), jnp.float32)
    for ri in range(r):
        for b in range(nb):
            Wbd = Wbd.at[ri, b * bin_:(b + 1) * bin_, b * bout:(b + 1) * bout].set(w[ri, b])
    rootp = jnp.zeros((k_pad, D), jnp.float32).at[:root.shape[0], :root.shape[1]].set(root)
    Wext = jnp.concatenate([Wbd, rootp[None]], axis=0)
    bext = jnp.zeros((RR, 2, DH), jnp.float32)
    bext = bext.at[R].set(jnp.pad(bias, (0, D - bias.shape[0])).reshape(2, DH))
    return Wext, bext


def kernel(x, edge_index, edge_type, W1, b1, w1, root1, bias1, w2, root2, bias2):
    src = edge_index[0]
    dst = edge_index[1]
    typ = edge_type

    h1 = _linear(x, W1, b1, bm=1000)

    Wext1, bext1 = _build_weights(w1, root1, bias1, k_pad=500)
    Hcat1 = _hcat(h1, Wext1, bext1, relu_in=False)

    invw = _count_kernel(dst, typ)

    Y1 = _agg_kernel(Hcat1, src, dst, typ, invw)
    h2 = jnp.concatenate([Y1[:N], Y1[N:]], axis=1)

    Wext2, bext2 = _build_weights(w2, root2, bias2, k_pad=D)
    Hcat2 = _hcat(h2, Wext2, bext2, relu_in=True)

    Y2 = _agg_kernel(Hcat2, src, dst, typ, invw)
    return jnp.concatenate([Y2[:N], Y2[N:]], axis=1)[:, :250]
